# baseline (speedup 1.0000x reference)
import functools

import jax
import jax.numpy as jnp
import numpy as np
from jax import lax
from jax.experimental import pallas as pl
from jax.experimental.pallas import tpu as pltpu
from jax.experimental.pallas import tpu_sc as plsc

B, L, D = 4096, 200, 64
N_IDX = B * L
IDX_COLS = 128
IDX_ROWS = N_IDX // IDX_COLS
NC, NS = 2, 16
NW = NC * NS
W_B = B // NW
W_IDX_ROWS = IDX_ROWS // NW

MAG = float(np.float32(5.0) / np.sqrt(np.float32(L * D)))

N_ELEM = N_IDX * D
ROWS128 = N_ELEM // 128
BLK = 2048
ROWS_PER_B = L * D // 128

CHUNK_IR = 4
CHUNK_ROWS = CHUNK_IR * IDX_COLS
N_CHUNKS = W_IDX_ROWS // CHUNK_IR
C128 = CHUNK_ROWS * D // 128


def _sc_reformat_ids(input_ids):
    mesh = plsc.VectorSubcoreMesh(core_axis_name="c", subcore_axis_name="s")

    @functools.partial(
        pl.kernel,
        mesh=mesh,
        compiler_params=pltpu.CompilerParams(use_tc_tiling_on_sc=True),
        out_type=jax.ShapeDtypeStruct((N_IDX,), jnp.int32),
        scratch_types=[
            pltpu.VMEM((W_B, L), jnp.int32),
            pltpu.VMEM((W_B * L,), jnp.int32),
        ],
    )
    def k(ids_hbm, out_hbm, in_v, out_v):
        wid = lax.axis_index("s") * NC + lax.axis_index("c")
        b0 = wid * W_B
        pltpu.sync_copy(ids_hbm.at[pl.ds(b0, W_B)], in_v)

        def row(r, _):
            base = r * L
            for c in range(12):
                out_v[pl.ds(base + c * 16, 16)] = in_v[r, pl.ds(c * 16, 16)]
            out_v[pl.ds(base + 184, 16)] = in_v[r, pl.ds(184, 16)]
            return _

        lax.fori_loop(0, W_B, row, None)
        pltpu.sync_copy(out_v, out_hbm.at[pl.ds(b0 * L, W_B * L)])

    return k(input_ids)


TBL_V = 1000000
TBL_CHUNK = 400
TBL_NCH = TBL_V // TBL_CHUNK
TBL_CPW = -(-TBL_NCH // NW)


def _sc_depad_table(table):
    mesh = plsc.VectorSubcoreMesh(core_axis_name="c", subcore_axis_name="s")

    @functools.partial(
        pl.kernel,
        mesh=mesh,
        compiler_params=pltpu.CompilerParams(use_tc_tiling_on_sc=True),
        out_type=jax.ShapeDtypeStruct((TBL_V * D,), jnp.float32),
        scratch_types=[
            pltpu.VMEM((TBL_CHUNK, D), jnp.float32),
            pltpu.VMEM((TBL_CHUNK * D,), jnp.float32),
        ],
    )
    def k(tbl_hbm, out_hbm, in_v, out_v):
        wid = lax.axis_index("s") * NC + lax.axis_index("c")

        def body(kk, _):
            g = wid + NW * kk

            @pl.when(g < TBL_NCH)
            def _do():
                pltpu.sync_copy(tbl_hbm.at[pl.ds(g * TBL_CHUNK, TBL_CHUNK)], in_v)

                def row(r, carry):
                    for c in range(D // 16):
                        out_v[pl.ds(r * D + c * 16, 16)] = in_v[r, pl.ds(c * 16, 16)]
                    return carry

                lax.fori_loop(0, TBL_CHUNK, row, None)
                pltpu.sync_copy(
                    out_v, out_hbm.at[pl.ds(g * TBL_CHUNK * D, TBL_CHUNK * D)]
                )

            return _

        lax.fori_loop(0, TBL_CPW, body, None)

    return k(table)


def _sc_gather(table, idx_flat):
    mesh = plsc.VectorSubcoreMesh(core_axis_name="c", subcore_axis_name="s")

    @functools.partial(
        pl.kernel,
        mesh=mesh,
        compiler_params=pltpu.CompilerParams(use_tc_tiling_on_sc=False),
        out_type=jax.ShapeDtypeStruct((ROWS128, 128), jnp.float32),
        scratch_types=[
            pltpu.VMEM((CHUNK_ROWS,), jnp.int32),
            pltpu.VMEM((CHUNK_ROWS, D), jnp.float32),
            pltpu.VMEM((C128, 128), jnp.float32),
            pltpu.SemaphoreType.DMA,
        ],
    )
    def k(table_hbm, idx_hbm, out_hbm, idx_v, rows_v, rows128_v, sem):
        wid = lax.axis_index("s") * NC + lax.axis_index("c")
        base_ir = wid * W_IDX_ROWS

        def body(c, _):
            ir = base_ir + c * CHUNK_IR
            pltpu.sync_copy(
                idx_hbm.at[pl.ds(ir * IDX_COLS, CHUNK_ROWS)], idx_v
            )
            cps = [
                pltpu.async_copy(
                    table_hbm.at[idx_v.at[pl.ds(j * IDX_COLS, IDX_COLS)]],
                    rows_v.at[pl.ds(j * IDX_COLS, IDX_COLS)],
                    sem,
                )
                for j in range(CHUNK_IR)
            ]
            for cp in cps:
                cp.wait()

            def repack(q, _):
                for h in range(2):
                    for cc in range(4):
                        rows128_v[q, pl.ds(h * 64 + cc * 16, 16)] = (
                            rows_v[2 * q + h, pl.ds(cc * 16, 16)]
                        )
                return _

            lax.fori_loop(0, C128, repack, None)
            pltpu.sync_copy(
                rows128_v, out_hbm.at[pl.ds(ir * IDX_COLS * D // 128, C128)]
            )
            return _

        lax.fori_loop(0, N_CHUNKS, body, None)

    return k(table, idx_flat)


def _sc_add(e128, n128):
    mesh = plsc.VectorSubcoreMesh(core_axis_name="c", subcore_axis_name="s")

    @functools.partial(
        pl.kernel,
        mesh=mesh,
        compiler_params=pltpu.CompilerParams(use_tc_tiling_on_sc=True),
        out_type=jax.ShapeDtypeStruct((B, L, D), jnp.float32),
        scratch_types=[
            pltpu.VMEM((2, 2 * ROWS_PER_B, 128), jnp.float32),
            pltpu.VMEM((2 * ROWS_PER_B, 128), jnp.float32),
            pltpu.VMEM((2, L, D), jnp.float32),
            pltpu.SemaphoreType.DMA,
            pltpu.SemaphoreType.DMA,
            pltpu.SemaphoreType.DMA,
        ],
    )
    def k(e_hbm, n_hbm, out_hbm, e_v, n_v, acc_v, se, sn, so):
        wid = lax.axis_index("s") * NC + lax.axis_index("c")
        base_b = wid * W_B
        nch = W_B // 2

        def e_in(c, buf):
            r0 = (base_b + 2 * c) * ROWS_PER_B
            return pltpu.async_copy(
                e_hbm.at[pl.ds(r0, 2 * ROWS_PER_B)], e_v.at[buf], se
            )

        def n_in(c):
            r0 = (base_b + 2 * c) * ROWS_PER_B
            return pltpu.async_copy(
                n_hbm.at[pl.ds(r0, 2 * ROWS_PER_B)], n_v, sn
            )

        def out_cp(c, h):
            b0 = base_b + 2 * c
            return pltpu.make_async_copy(
                acc_v.at[pl.ds(h, 1)], out_hbm.at[pl.ds(b0 + h, 1)], so
            )

        def addhalf(buf, h):
            def addrow(q, carry):
                ll = 2 * q
                for hh in range(2):
                    for cc in range(4):
                        acc_v[h, ll + hh, pl.ds(cc * 16, 16)] = (
                            e_v[buf, h * ROWS_PER_B + q, pl.ds(hh * 64 + cc * 16, 16)]
                            + n_v[h * ROWS_PER_B + q, pl.ds(hh * 64 + cc * 16, 16)]
                        )
                return carry

            lax.fori_loop(0, ROWS_PER_B, addrow, None)

        cp = e_in(0, 0)
        n0 = n_in(0)
        cp.wait()
        n0.wait()

        def body(c, _):
            buf = lax.rem(c, 2)

            @pl.when(c > 0)
            def _drain():
                out_cp(c - 1, 0).wait()
                out_cp(c - 1, 1).wait()

            addhalf(buf, 0)
            out_cp(c, 0).start()

            @pl.when(c + 1 < nch)
            def _pref():
                e_in(c + 1, 1 - buf)

            addhalf(buf, 1)
            out_cp(c, 1).start()

            @pl.when(c + 1 < nch)
            def _nn():
                n_in(c + 1)

            @pl.when(c + 1 < nch)
            def _wt():
                pltpu.make_async_copy(
                    e_hbm.at[pl.ds((base_b + 2 * (c + 1)) * ROWS_PER_B, 2 * ROWS_PER_B)],
                    e_v.at[1 - buf],
                    se,
                ).wait()
                pltpu.make_async_copy(
                    n_hbm.at[pl.ds((base_b + 2 * (c + 1)) * ROWS_PER_B, 2 * ROWS_PER_B)],
                    n_v,
                    sn,
                ).wait()

            return _

        lax.fori_loop(0, nch, body, None)
        out_cp(nch - 1, 0).wait()
        out_cp(nch - 1, 1).wait()

    return k(e128, n128)


def _threefry_noise(shape, base):
    it = (
        lax.broadcasted_iota(jnp.int32, shape, 0) * shape[1]
        + lax.broadcasted_iota(jnp.int32, shape, 1)
    ).astype(jnp.uint32)
    x1 = base.astype(jnp.uint32) + it

    k1 = jnp.uint32(42)
    k2 = jnp.uint32(0x1BD11BDA ^ 42)

    def rotl(v, r):
        return (v << jnp.uint32(r)) | (v >> jnp.uint32(32 - r))

    xb = x1 + k1
    xa = xb
    xb = rotl(xb, 13)
    xb = xa ^ xb
    for r in (15, 26, 6):
        xa = xa + xb
        xb = rotl(xb, r)
        xb = xa ^ xb
    xa = xa + k1
    xb = xb + (k2 + jnp.uint32(1))
    ks = (k1, k2, jnp.uint32(0))
    rots = ((17, 29, 16, 24), (13, 15, 26, 6))
    for i in range(1, 5):
        for r in rots[0]:
            xa = xa + xb
            xb = rotl(xb, r)
            xb = xa ^ xb
        xa = xa + ks[1]
        xb = xb + (ks[2] + jnp.uint32(i + 1))
        ks = (ks[1], ks[2], ks[0])
        rots = (rots[1], rots[0])
    bits = xa ^ xb

    uf = lax.bitcast_convert_type(
        (bits >> jnp.uint32(9)) | jnp.uint32(0x3F800000), jnp.float32
    )
    u = uf - jnp.float32(1.0)
    r2 = u * jnp.float32(2.0) - jnp.float32(1.0)
    return r2 * jnp.float32(MAG)


def _noise_body(o_ref):
    pid = pl.program_id(0)
    base = pid * (BLK * 128)
    o_ref[...] = _threefry_noise((BLK, 128), jnp.int32(0) + base)


def _tc_noise():
    return pl.pallas_call(
        _noise_body,
        grid=(ROWS128 // BLK,),
        out_specs=pl.BlockSpec((BLK, 128), lambda i: (i, 0)),
        out_shape=jax.ShapeDtypeStruct((ROWS128, 128), jnp.float32),
    )()


def kernel(input_ids, table):
    n128 = _tc_noise()
    ids_flat = _sc_reformat_ids(input_ids.astype(jnp.int32))
    tbl_flat = _sc_depad_table(table)
    e128 = _sc_gather(tbl_flat.reshape(TBL_V, D), ids_flat)
    return _sc_add(e128, n128)


# speedup vs baseline: 1.3693x; 1.3693x over previous
import functools

import jax
import jax.numpy as jnp
import numpy as np
from jax import lax
from jax.experimental import pallas as pl
from jax.experimental.pallas import tpu as pltpu
from jax.experimental.pallas import tpu_sc as plsc

B, L, D = 4096, 200, 64
N_IDX = B * L
IDX_COLS = 128
IDX_ROWS = N_IDX // IDX_COLS
NC, NS = 2, 16
NW = NC * NS
W_IDX_ROWS = IDX_ROWS // NW
CHUNK_IR = 4
CHUNK_ROWS = CHUNK_IR * IDX_COLS
N_CHUNKS = W_IDX_ROWS // CHUNK_IR

MAG = float(np.float32(5.0) / np.sqrt(np.float32(L * D)))

N_ELEM = N_IDX * D
ROWS128 = N_ELEM // 128
BLK = 2048


def _sc_gather(table, idx2d):
    mesh = plsc.VectorSubcoreMesh(core_axis_name="c", subcore_axis_name="s")

    @functools.partial(
        pl.kernel,
        mesh=mesh,
        compiler_params=pltpu.CompilerParams(use_tc_tiling_on_sc=False),
        out_type=jax.ShapeDtypeStruct((N_IDX, D), jnp.float32),
        scratch_types=[
            pltpu.VMEM((CHUNK_IR, IDX_COLS), jnp.int32),
            pltpu.VMEM((CHUNK_ROWS, D), jnp.float32),
            pltpu.SemaphoreType.DMA,
        ],
    )
    def k(table_hbm, idx_hbm, out_hbm, idx_v, rows_v, sem):
        wid = lax.axis_index("s") * NC + lax.axis_index("c")
        base_ir = wid * W_IDX_ROWS

        def body(c, _):
            ir = base_ir + c * CHUNK_IR
            pltpu.sync_copy(idx_hbm.at[pl.ds(ir, CHUNK_IR)], idx_v)
            cps = [
                pltpu.async_copy(
                    table_hbm.at[idx_v.at[j]],
                    rows_v.at[pl.ds(j * IDX_COLS, IDX_COLS)],
                    sem,
                )
                for j in range(CHUNK_IR)
            ]
            for cp in cps:
                cp.wait()
            pltpu.sync_copy(rows_v, out_hbm.at[pl.ds(ir * IDX_COLS, CHUNK_ROWS)])
            return _

        lax.fori_loop(0, N_CHUNKS, body, None)

    return k(table, idx2d)


def _threefry_noise(shape, base):
    it = (
        lax.broadcasted_iota(jnp.int32, shape, 0) * shape[1]
        + lax.broadcasted_iota(jnp.int32, shape, 1)
    ).astype(jnp.uint32)
    x1 = base.astype(jnp.uint32) + it

    k1 = jnp.uint32(42)
    k2 = jnp.uint32(0x1BD11BDA ^ 42)

    def rotl(v, r):
        return (v << jnp.uint32(r)) | (v >> jnp.uint32(32 - r))

    xb = x1 + k1
    xa = xb
    xb = rotl(xb, 13)
    xb = xa ^ xb
    for r in (15, 26, 6):
        xa = xa + xb
        xb = rotl(xb, r)
        xb = xa ^ xb
    xa = xa + k1
    xb = xb + (k2 + jnp.uint32(1))
    ks = (k1, k2, jnp.uint32(0))
    rots = ((17, 29, 16, 24), (13, 15, 26, 6))
    for i in range(1, 5):
        for r in rots[0]:
            xa = xa + xb
            xb = rotl(xb, r)
            xb = xa ^ xb
        xa = xa + ks[1]
        xb = xb + (ks[2] + jnp.uint32(i + 1))
        ks = (ks[1], ks[2], ks[0])
        rots = (rots[1], rots[0])
    bits = xa ^ xb

    uf = lax.bitcast_convert_type(
        (bits >> jnp.uint32(9)) | jnp.uint32(0x3F800000), jnp.float32
    )
    u = uf - jnp.float32(1.0)
    r2 = u * jnp.float32(2.0) - jnp.float32(1.0)
    return r2 * jnp.float32(MAG)


def _noise_add_body(x_ref, o_ref):
    pid = pl.program_id(0)
    base = pid * (BLK * 128)
    o_ref[...] = x_ref[...] + _threefry_noise((BLK, 128), jnp.int32(0) + base)


def _tc_noise_add(e128):
    return pl.pallas_call(
        _noise_add_body,
        grid=(ROWS128 // BLK,),
        in_specs=[pl.BlockSpec((BLK, 128), lambda i: (i, 0))],
        out_specs=pl.BlockSpec((BLK, 128), lambda i: (i, 0)),
        out_shape=jax.ShapeDtypeStruct((ROWS128, 128), jnp.float32),
    )(e128)


def kernel(input_ids, table):
    ids = input_ids.reshape(IDX_ROWS, IDX_COLS).astype(jnp.int32)
    embeds = _sc_gather(table, ids)
    out128 = _tc_noise_add(embeds.reshape(ROWS128, 128))
    return out128.reshape(B, L, D)
